# Initial kernel scaffold; baseline (speedup 1.0000x reference)
import jax
import jax.numpy as jnp
from jax.experimental import pallas as pl


def kernel(i, j, table_i, table_j):
    raise NotImplementedError("write your pallas kernel here")


# speedup vs baseline: 4.2722x; 4.2722x over previous
import functools

import jax
import jax.numpy as jnp
from jax.experimental import pallas as pl
from jax.experimental.pallas import tpu as pltpu

B, T, C = 4, 224, 128
H, W = 224, 224
HB = 8


def _body(i_ref, j_ref, ti_ref, tj_ref, out_ref):
    def slab(idx_2d, table):
        flat = idx_2d.reshape(1, HB * W)
        iota = jax.lax.broadcasted_iota(jnp.int32, (T, HB * W), 0)
        onehot = (iota == flat).astype(jnp.float32)
        res = jax.lax.dot_general(
            table, onehot, (((0,), (0,)), ((), ())),
            preferred_element_type=jnp.float32,
        )
        return res.reshape(C, HB, W)

    out_ref[0, 0:C] = slab(i_ref[0], ti_ref[...])
    out_ref[0, C : 2 * C] = slab(j_ref[0], tj_ref[...])


@functools.partial(jax.jit, static_argnames=())
def kernel(i, j, table_i, table_j):
    grid = (B, H // HB)
    idx_spec = pl.BlockSpec((1, HB, W), lambda b, h: (b, h, 0))
    tab_spec = pl.BlockSpec((T, C), lambda b, h: (0, 0))
    out_spec = pl.BlockSpec((1, 2 * C, HB, W), lambda b, h: (b, 0, h, 0))
    return pl.pallas_call(
        _body,
        grid=grid,
        in_specs=[idx_spec, idx_spec, tab_spec, tab_spec],
        out_specs=out_spec,
        out_shape=jax.ShapeDtypeStruct((B, 2 * C, H, W), jnp.float32),
        compiler_params=pltpu.CompilerParams(
            dimension_semantics=("parallel", "parallel"),
        ),
    )(i, j, table_i, table_j)


# speedup vs baseline: 4.6828x; 1.0961x over previous
import functools

import jax
import jax.numpy as jnp
from jax.experimental import pallas as pl
from jax.experimental.pallas import tpu as pltpu

B, T, C = 4, 224, 128
H, W = 224, 224
HB = 16


def _body(i_ref, j_ref, ti_ref, tj_ref, out_ref):
    def slab(idx_2d, table):
        flat = idx_2d.reshape(1, HB * W)
        iota = jax.lax.broadcasted_iota(jnp.int32, (T, HB * W), 0)
        onehot = (iota == flat).astype(jnp.float32)
        res = jax.lax.dot_general(
            table, onehot, (((0,), (0,)), ((), ())),
            preferred_element_type=jnp.float32,
        )
        return res.reshape(C, HB, W)

    out_ref[0, 0:C] = slab(i_ref[0], ti_ref[...])
    out_ref[0, C : 2 * C] = slab(j_ref[0], tj_ref[...])


@functools.partial(jax.jit, static_argnames=())
def kernel(i, j, table_i, table_j):
    grid = (B, H // HB)
    idx_spec = pl.BlockSpec((1, HB, W), lambda b, h: (b, h, 0))
    tab_spec = pl.BlockSpec((T, C), lambda b, h: (0, 0))
    out_spec = pl.BlockSpec((1, 2 * C, HB, W), lambda b, h: (b, 0, h, 0))
    return pl.pallas_call(
        _body,
        grid=grid,
        in_specs=[idx_spec, idx_spec, tab_spec, tab_spec],
        out_specs=out_spec,
        out_shape=jax.ShapeDtypeStruct((B, 2 * C, H, W), jnp.float32),
        compiler_params=pltpu.CompilerParams(
            dimension_semantics=("parallel", "parallel"),
        ),
    )(i, j, table_i, table_j)
